# Initial kernel scaffold; baseline (speedup 1.0000x reference)
import functools

import jax
import jax.numpy as jnp
from jax import lax
from jax.experimental import pallas as pl
from jax.experimental.pallas import tpu as pltpu
from jax.experimental.pallas import tpu_sc as plsc

N = 10000
D = 128
L = 6
B = 16
MAX_Z = 100

_info = plsc.get_sparse_core_info()
NC = _info.num_cores
NS = _info.num_subcores
NW = NC * NS
GRP = 128

ACC_ROWS = 10112
TRASH = N


def _make_agg(F: int, groups: int):
    mesh = plsc.VectorSubcoreMesh(core_axis_name="c", subcore_axis_name="s")
    rows_per_tile = ACC_ROWS // NS

    @functools.partial(
        pl.kernel,
        mesh=mesh,
        out_type=jax.ShapeDtypeStruct((NC, ACC_ROWS, F), jnp.float32),
        scratch_types=[
            pltpu.VMEM((groups, GRP), jnp.int32),
            pltpu.VMEM((groups, GRP), jnp.int32),
            pltpu.VMEM((GRP, F), jnp.float32),
            pltpu.VMEM((GRP, F), jnp.float32),
            pltpu.VMEM_SHARED((ACC_ROWS, F), jnp.float32),
            pltpu.SemaphoreType.DMA,
            pltpu.SemaphoreType.DMA,
        ],
    )
    def agg(table, srcs, dsts, zeros, out, isrc, idst, buf0, buf1, acc, s0, s1):
        c = lax.axis_index("c")
        s = lax.axis_index("s")
        wid = s * NC + c
        lo = s * rows_per_tile
        pltpu.sync_copy(zeros.at[pl.ds(lo, rows_per_tile)],
                        acc.at[pl.ds(lo, rows_per_tile)])
        pltpu.sync_copy(srcs.at[wid], isrc)
        pltpu.sync_copy(dsts.at[wid], idst)
        plsc.subcore_barrier()

        def pair(i, carry):
            g = 2 * i
            c0 = pltpu.async_copy(table.at[isrc.at[g]], buf0, s0)
            c1 = pltpu.async_copy(table.at[isrc.at[g + 1]], buf1, s1)
            c0.wait()
            pltpu.sync_copy(buf0, acc.at[idst.at[g]], add=True)
            c1.wait()
            pltpu.sync_copy(buf1, acc.at[idst.at[g + 1]], add=True)
            return carry

        lax.fori_loop(0, groups // 2, pair, 0)
        plsc.subcore_barrier()
        pltpu.sync_copy(acc.at[pl.ds(lo, rows_per_tile)],
                        out.at[c, pl.ds(lo, rows_per_tile)])

    return agg


def _tc0_body(degs_ref, z_ref, emb_ref, W0_ref, p0_ref, dinv_ref):
    deg = (degs_ref[0, :N, 0:1] + degs_ref[1, :N, 0:1]) + 1.0
    dinv = lax.rsqrt(deg)
    dinv_ref[...] = dinv
    T0 = jnp.dot(emb_ref[...], W0_ref[...],
                 preferred_element_type=jnp.float32)
    z = z_ref[...]
    cls = lax.broadcasted_iota(jnp.int32, (1, MAX_Z), 1)
    oh = (z == cls).astype(jnp.float32)
    h0 = jnp.dot(oh, T0, preferred_element_type=jnp.float32)
    p0_ref[...] = h0 * dinv


def _tc_layer_body(k, S_ref, pprev_ref, dinv_ref, Ws_ref, bs_ref, pk_ref):
    dinv = dinv_ref[...]
    agg = S_ref[0, :N, :] + S_ref[1, :N, :] + pprev_ref[...]
    x = jnp.maximum(dinv * agg + bs_ref[k - 1:k, :], 0.0)
    pk_ref[...] = dinv * jnp.dot(x, Ws_ref[k],
                                 preferred_element_type=jnp.float32)


def _tc_final_body(S_ref, p5_ref, dinv_ref, bs_ref, batch_ref,
                   W1_ref, b1_ref, W2_ref, b2_ref, out_ref):
    dinv = dinv_ref[...]
    agg = S_ref[0, :N, :] + S_ref[1, :N, :] + p5_ref[...]
    x = jnp.maximum(dinv * agg + bs_ref[L - 1:L, :], 0.0)
    seg = lax.broadcasted_iota(jnp.int32, (B, 1), 0)
    ohb = (seg == batch_ref[...]).astype(jnp.float32)
    g = jnp.dot(ohb, x, preferred_element_type=jnp.float32)
    h = jnp.maximum(jnp.dot(g, W1_ref[...],
                            preferred_element_type=jnp.float32)
                    + b1_ref[...], 0.0)
    out_ref[...] = (jnp.dot(h, W2_ref[...],
                            preferred_element_type=jnp.float32)
                    + b2_ref[0:1, 0:1])


def kernel(z, edge_index, batch, emb, Ws, bs, W1, b1, W2, b2):
    E = edge_index.shape[1]
    per_w = -(-E // (NW * 2 * GRP)) * (2 * GRP)
    groups = per_w // GRP
    e_pad = per_w * NW

    ei = edge_index.astype(jnp.int32)
    pad = e_pad - E
    src_p = jnp.concatenate([ei[0], jnp.zeros((pad,), jnp.int32)])
    dst_p = jnp.concatenate([ei[1], jnp.full((pad,), TRASH, jnp.int32)])
    srcs = src_p.reshape(NW, groups, GRP)
    dsts = dst_p.reshape(NW, groups, GRP)

    zeros128 = jnp.zeros((ACC_ROWS, D), jnp.float32)
    zeros16 = jnp.zeros((ACC_ROWS, 16), jnp.float32)
    ones16 = jnp.ones((N, 16), jnp.float32)

    agg128 = _make_agg(D, groups)
    agg16 = _make_agg(16, groups)

    degs = agg16(ones16, srcs, dsts, zeros16)

    z2 = z.astype(jnp.int32).reshape(N, 1)
    p0, dinv = pl.pallas_call(
        _tc0_body,
        out_shape=(jax.ShapeDtypeStruct((N, D), jnp.float32),
                   jax.ShapeDtypeStruct((N, 1), jnp.float32)),
    )(degs, z2, emb, Ws[0])

    p = p0
    for k in range(1, L + 1):
        S = agg128(p, srcs, dsts, zeros128)
        if k < L:
            p = pl.pallas_call(
                functools.partial(_tc_layer_body, k),
                out_shape=jax.ShapeDtypeStruct((N, D), jnp.float32),
            )(S, p, dinv, Ws, bs)
        else:
            out = pl.pallas_call(
                _tc_final_body,
                out_shape=jax.ShapeDtypeStruct((B, 1), jnp.float32),
            )(S, p, dinv, bs, batch.astype(jnp.int32).reshape(1, N),
              W1, b1.reshape(1, D), W2, b2.reshape(1, 1))

    return out.reshape(-1)


# speedup vs baseline: 5.6265x; 5.6265x over previous
import functools

import jax
import jax.numpy as jnp
from jax import lax
from jax.experimental import pallas as pl
from jax.experimental.pallas import tpu as pltpu
from jax.experimental.pallas import tpu_sc as plsc

N = 10000
D = 128
L = 6
B = 16
MAX_Z = 100

_info = plsc.get_sparse_core_info()
NC = _info.num_cores
NS = _info.num_subcores
NW = NC * NS
GRP = 128

ACC_ROWS = 10112
TRASH = N


def _make_agg(groups: int, gather: bool):
    mesh = plsc.VectorSubcoreMesh(core_axis_name="c", subcore_axis_name="s")
    rows_per_tile = ACC_ROWS // NS
    IC = 8
    chunks = groups // IC

    @functools.partial(
        pl.kernel,
        mesh=mesh,
        out_type=jax.ShapeDtypeStruct((NC, ACC_ROWS, D), jnp.float32),
        scratch_types=[
            pltpu.VMEM((IC, GRP), jnp.int32),
            pltpu.VMEM((IC, GRP), jnp.int32),
            pltpu.VMEM((GRP, D), jnp.float32),
            pltpu.VMEM((GRP, D), jnp.float32),
            pltpu.VMEM_SHARED((ACC_ROWS, D), jnp.float32),
            pltpu.SemaphoreType.DMA,
            pltpu.SemaphoreType.DMA,
        ],
    )
    def agg(table, srcs, dsts, zeros, out, isrc, idst, buf0, buf1, acc, s0, s1):
        c = lax.axis_index("c")
        s = lax.axis_index("s")
        wid = s * NC + c
        lo = s * rows_per_tile
        pltpu.sync_copy(zeros.at[pl.ds(lo, rows_per_tile)],
                        acc.at[pl.ds(lo, rows_per_tile)])
        if not gather:
            pltpu.sync_copy(table, buf0)
        plsc.subcore_barrier()

        def chunk(i, carry):
            pltpu.sync_copy(dsts.at[wid, pl.ds(i * IC, IC)], idst)
            if gather:
                pltpu.sync_copy(srcs.at[wid, pl.ds(i * IC, IC)], isrc)
                for j in range(0, IC, 2):
                    c0 = pltpu.async_copy(table.at[isrc.at[j]], buf0, s0)
                    c1 = pltpu.async_copy(table.at[isrc.at[j + 1]], buf1, s1)
                    c0.wait()
                    pltpu.sync_copy(buf0, acc.at[idst.at[j]], add=True)
                    c1.wait()
                    pltpu.sync_copy(buf1, acc.at[idst.at[j + 1]], add=True)
            else:
                for j in range(IC):
                    pltpu.sync_copy(buf0, acc.at[idst.at[j]], add=True)
            return carry

        lax.fori_loop(0, chunks, chunk, 0)
        plsc.subcore_barrier()
        pltpu.sync_copy(acc.at[pl.ds(lo, rows_per_tile)],
                        out.at[c, pl.ds(lo, rows_per_tile)])

    return agg


def _tc0_body(degs_ref, z_ref, emb_ref, W0_ref, p0_ref, dinv_ref):
    deg = (degs_ref[0, :N, 0:1] + degs_ref[1, :N, 0:1]) + 1.0
    dinv = lax.rsqrt(deg)
    dinv_ref[...] = dinv
    T0 = jnp.dot(emb_ref[...], W0_ref[...],
                 preferred_element_type=jnp.float32)
    z = z_ref[...]
    cls = lax.broadcasted_iota(jnp.int32, (1, MAX_Z), 1)
    oh = (z == cls).astype(jnp.float32)
    h0 = jnp.dot(oh, T0, preferred_element_type=jnp.float32)
    p0_ref[...] = h0 * dinv


def _tc_layer_body(k, S_ref, pprev_ref, dinv_ref, Ws_ref, bs_ref, pk_ref):
    dinv = dinv_ref[...]
    agg = S_ref[0, :N, :] + S_ref[1, :N, :] + pprev_ref[...]
    x = jnp.maximum(dinv * agg + bs_ref[k - 1:k, :], 0.0)
    pk_ref[...] = dinv * jnp.dot(x, Ws_ref[k],
                                 preferred_element_type=jnp.float32)


def _tc_final_body(S_ref, p5_ref, dinv_ref, bs_ref, batch_ref,
                   W1_ref, b1_ref, W2_ref, b2_ref, out_ref):
    dinv = dinv_ref[...]
    agg = S_ref[0, :N, :] + S_ref[1, :N, :] + p5_ref[...]
    x = jnp.maximum(dinv * agg + bs_ref[L - 1:L, :], 0.0)
    seg = lax.broadcasted_iota(jnp.int32, (B, 1), 0)
    ohb = (seg == batch_ref[...]).astype(jnp.float32)
    g = jnp.dot(ohb, x, preferred_element_type=jnp.float32)
    h = jnp.maximum(jnp.dot(g, W1_ref[...],
                            preferred_element_type=jnp.float32)
                    + b1_ref[...], 0.0)
    out_ref[...] = (jnp.dot(h, W2_ref[...],
                            preferred_element_type=jnp.float32)
                    + b2_ref[0:1, 0:1])


def kernel(z, edge_index, batch, emb, Ws, bs, W1, b1, W2, b2):
    E = edge_index.shape[1]
    per_w = -(-E // (NW * 8 * GRP)) * (8 * GRP)
    groups = per_w // GRP
    e_pad = per_w * NW

    ei = edge_index.astype(jnp.int32)
    pad = e_pad - E
    src_p = jnp.concatenate([ei[0], jnp.zeros((pad,), jnp.int32)])
    dst_p = jnp.concatenate([ei[1], jnp.full((pad,), TRASH, jnp.int32)])
    srcs = src_p.reshape(NW, groups, GRP)
    dsts = dst_p.reshape(NW, groups, GRP)

    zeros128 = jnp.zeros((ACC_ROWS, D), jnp.float32)
    ones_blk = jnp.ones((GRP, D), jnp.float32)

    agg128 = _make_agg(groups, gather=True)
    deg_cnt = _make_agg(groups, gather=False)

    degs = deg_cnt(ones_blk, srcs, dsts, zeros128)

    z2 = z.astype(jnp.int32).reshape(N, 1)
    p0, dinv = pl.pallas_call(
        _tc0_body,
        out_shape=(jax.ShapeDtypeStruct((N, D), jnp.float32),
                   jax.ShapeDtypeStruct((N, 1), jnp.float32)),
    )(degs, z2, emb, Ws[0])

    p = p0
    for k in range(1, L + 1):
        S = agg128(p, srcs, dsts, zeros128)
        if k < L:
            p = pl.pallas_call(
                functools.partial(_tc_layer_body, k),
                out_shape=jax.ShapeDtypeStruct((N, D), jnp.float32),
            )(S, p, dinv, Ws, bs)
        else:
            out = pl.pallas_call(
                _tc_final_body,
                out_shape=jax.ShapeDtypeStruct((B, 1), jnp.float32),
            )(S, p, dinv, bs, batch.astype(jnp.int32).reshape(1, N),
              W1, b1.reshape(1, D), W2, b2.reshape(1, 1))

    return out.reshape(-1)


# speedup vs baseline: 17.9401x; 3.1885x over previous
import functools

import jax
import jax.numpy as jnp
from jax import lax
from jax.experimental import pallas as pl
from jax.experimental.pallas import tpu as pltpu
from jax.experimental.pallas import tpu_sc as plsc

N = 10000
D = 128
L = 6
B = 16
MAX_Z = 100

_info = plsc.get_sparse_core_info()
NC = _info.num_cores
NS = _info.num_subcores
NW = NC * NS
GRP = 128

ACC_ROWS = 10112
TRASH = N


def _make_agg(groups: int, gather: bool):
    mesh = plsc.VectorSubcoreMesh(core_axis_name="c", subcore_axis_name="s")
    rows_per_tile = ACC_ROWS // NS
    IC = 16
    chunks = groups // IC

    @functools.partial(
        pl.kernel,
        mesh=mesh,
        out_type=jax.ShapeDtypeStruct((NC, ACC_ROWS, D), jnp.float32),
        scratch_types=[
            pltpu.VMEM((IC, GRP), jnp.int32),
            pltpu.VMEM((IC, GRP), jnp.int32),
            pltpu.VMEM((GRP, D), jnp.float32),
            pltpu.VMEM((GRP, D), jnp.float32),
            pltpu.SemaphoreType.DMA,
            pltpu.SemaphoreType.DMA,
            pltpu.SemaphoreType.DMA,
            pltpu.SemaphoreType.DMA,
            pltpu.VMEM_SHARED((ACC_ROWS, D), jnp.float32),
        ],
    )
    def agg(table, srcs, dsts, zeros, out,
            isrc, idst, buf0, buf1, g0, g1, s0, s1, acc):
        c = lax.axis_index("c")
        s = lax.axis_index("s")
        wid = s * NC + c
        lo = s * rows_per_tile
        bufs = (buf0, buf1)
        gsems = (g0, g1)
        ssems = (s0, s1)
        pltpu.sync_copy(zeros.at[pl.ds(lo, rows_per_tile)],
                        acc.at[pl.ds(lo, rows_per_tile)])
        if not gather:
            pltpu.sync_copy(table, buf0)
        plsc.subcore_barrier()

        def chunk(i, carry):
            pltpu.sync_copy(dsts.at[wid, pl.ds(i * IC, IC)], idst)
            if gather:
                pltpu.sync_copy(srcs.at[wid, pl.ds(i * IC, IC)], isrc)
                gh = [None, None]
                sh = [None, None]
                gh[0] = pltpu.async_copy(table.at[isrc.at[0]], buf0, g0)
                for j in range(IC):
                    b = j & 1
                    gh[b].wait()
                    if j + 1 < IC:
                        if sh[1 - b] is not None:
                            sh[1 - b].wait()
                        gh[1 - b] = pltpu.async_copy(
                            table.at[isrc.at[j + 1]], bufs[1 - b], gsems[1 - b])
                    sh[b] = pltpu.async_copy(
                        bufs[b], acc.at[idst.at[j]], ssems[b], add=True)
                sh[0].wait()
                sh[1].wait()
            else:
                sh = [None, None]
                for j in range(IC):
                    b = j & 1
                    if sh[b] is not None:
                        sh[b].wait()
                    sh[b] = pltpu.async_copy(
                        buf0, acc.at[idst.at[j]], ssems[b], add=True)
                sh[0].wait()
                sh[1].wait()
            return carry

        lax.fori_loop(0, chunks, chunk, 0)
        plsc.subcore_barrier()
        pltpu.sync_copy(acc.at[pl.ds(lo, rows_per_tile)],
                        out.at[c, pl.ds(lo, rows_per_tile)])

    return agg


def _tc0_body(degs_ref, z_ref, emb_ref, W0_ref, p0_ref, dinv_ref):
    deg = (degs_ref[0, :N, 0:1] + degs_ref[1, :N, 0:1]) + 1.0
    dinv = lax.rsqrt(deg)
    dinv_ref[...] = dinv
    T0 = jnp.dot(emb_ref[...], W0_ref[...],
                 preferred_element_type=jnp.float32)
    z = z_ref[...]
    cls = lax.broadcasted_iota(jnp.int32, (1, MAX_Z), 1)
    oh = (z == cls).astype(jnp.float32)
    h0 = jnp.dot(oh, T0, preferred_element_type=jnp.float32)
    p0_ref[...] = h0 * dinv


def _tc_layer_body(k, S_ref, pprev_ref, dinv_ref, Ws_ref, bs_ref, pk_ref):
    dinv = dinv_ref[...]
    agg = S_ref[0, :N, :] + S_ref[1, :N, :] + pprev_ref[...]
    x = jnp.maximum(dinv * agg + bs_ref[k - 1:k, :], 0.0)
    pk_ref[...] = dinv * jnp.dot(x, Ws_ref[k],
                                 preferred_element_type=jnp.float32)


def _tc_final_body(S_ref, p5_ref, dinv_ref, bs_ref, batch_ref,
                   W1_ref, b1_ref, W2_ref, b2_ref, out_ref):
    dinv = dinv_ref[...]
    agg = S_ref[0, :N, :] + S_ref[1, :N, :] + p5_ref[...]
    x = jnp.maximum(dinv * agg + bs_ref[L - 1:L, :], 0.0)
    seg = lax.broadcasted_iota(jnp.int32, (B, 1), 0)
    ohb = (seg == batch_ref[...]).astype(jnp.float32)
    g = jnp.dot(ohb, x, preferred_element_type=jnp.float32)
    h = jnp.maximum(jnp.dot(g, W1_ref[...],
                            preferred_element_type=jnp.float32)
                    + b1_ref[...], 0.0)
    out_ref[...] = (jnp.dot(h, W2_ref[...],
                            preferred_element_type=jnp.float32)
                    + b2_ref[0:1, 0:1])


def kernel(z, edge_index, batch, emb, Ws, bs, W1, b1, W2, b2):
    E = edge_index.shape[1]
    per_w = -(-E // (NW * 16 * GRP)) * (16 * GRP)
    groups = per_w // GRP
    e_pad = per_w * NW

    ei = edge_index.astype(jnp.int32)
    pad = e_pad - E
    pad_i = jnp.arange(pad, dtype=jnp.int32)
    src_p = jnp.concatenate([ei[0], pad_i % N])
    dst_p = jnp.concatenate([ei[1], TRASH + pad_i % (ACC_ROWS - N)])
    srcs = src_p.reshape(NW, groups, GRP)
    dsts = dst_p.reshape(NW, groups, GRP)

    zeros128 = jnp.zeros((ACC_ROWS, D), jnp.float32)
    ones_blk = jnp.ones((GRP, D), jnp.float32)

    agg128 = _make_agg(groups, gather=True)
    deg_cnt = _make_agg(groups, gather=False)

    degs = deg_cnt(ones_blk, srcs, dsts, zeros128)

    z2 = z.astype(jnp.int32).reshape(N, 1)
    p0, dinv = pl.pallas_call(
        _tc0_body,
        out_shape=(jax.ShapeDtypeStruct((N, D), jnp.float32),
                   jax.ShapeDtypeStruct((N, 1), jnp.float32)),
    )(degs, z2, emb, Ws[0])

    p = p0
    for k in range(1, L + 1):
        S = agg128(p, srcs, dsts, zeros128)
        if k < L:
            p = pl.pallas_call(
                functools.partial(_tc_layer_body, k),
                out_shape=jax.ShapeDtypeStruct((N, D), jnp.float32),
            )(S, p, dinv, Ws, bs)
        else:
            out = pl.pallas_call(
                _tc_final_body,
                out_shape=jax.ShapeDtypeStruct((B, 1), jnp.float32),
            )(S, p, dinv, bs, batch.astype(jnp.int32).reshape(1, N),
              W1, b1.reshape(1, D), W2, b2.reshape(1, 1))

    return out.reshape(-1)


# speedup vs baseline: 18.8070x; 1.0483x over previous
import functools

import jax
import jax.numpy as jnp
from jax import lax
from jax.experimental import pallas as pl
from jax.experimental.pallas import tpu as pltpu
from jax.experimental.pallas import tpu_sc as plsc

N = 10000
D = 128
L = 6
B = 16
MAX_Z = 100

_info = plsc.get_sparse_core_info()
NC = _info.num_cores
NS = _info.num_subcores
NW = NC * NS
GRP = 64

ACC_ROWS = 10112
TRASH = N


def _make_agg(groups: int, gather: bool):
    mesh = plsc.VectorSubcoreMesh(core_axis_name="c", subcore_axis_name="s")
    rows_per_tile = ACC_ROWS // NS
    IC = 32
    NB = 4
    chunks = groups // IC

    @functools.partial(
        pl.kernel,
        mesh=mesh,
        out_type=jax.ShapeDtypeStruct((NC, ACC_ROWS, D), jnp.float32),
        scratch_types=[
            pltpu.VMEM((IC, GRP), jnp.int32),
            pltpu.VMEM((IC, GRP), jnp.int32),
        ] + [pltpu.VMEM((GRP, D), jnp.float32)] * NB
          + [pltpu.SemaphoreType.DMA] * (2 * NB)
          + [pltpu.VMEM_SHARED((ACC_ROWS, D), jnp.float32)],
    )
    def agg(table, srcs, dsts, zeros, out, isrc, idst, *rest):
        bufs = rest[:NB]
        gsems = rest[NB:2 * NB]
        ssems = rest[2 * NB:3 * NB]
        acc = rest[3 * NB]
        c = lax.axis_index("c")
        s = lax.axis_index("s")
        wid = s * NC + c
        lo = s * rows_per_tile
        pltpu.sync_copy(zeros.at[pl.ds(lo, rows_per_tile)],
                        acc.at[pl.ds(lo, rows_per_tile)])
        if not gather:
            pltpu.sync_copy(table, bufs[0])
        plsc.subcore_barrier()

        def chunk(i, carry):
            pltpu.sync_copy(dsts.at[wid, pl.ds(i * IC, IC)], idst)
            if gather:
                pltpu.sync_copy(srcs.at[wid, pl.ds(i * IC, IC)], isrc)
                gh = [None] * NB
                sh = [None] * NB
                gh[0] = pltpu.async_copy(table.at[isrc.at[0]], bufs[0],
                                         gsems[0])
                gh[1] = pltpu.async_copy(table.at[isrc.at[1]], bufs[1],
                                         gsems[1])
                for j in range(IC):
                    b = j % NB
                    gh[b].wait()
                    if j + 2 < IC:
                        nb = (j + 2) % NB
                        if sh[nb] is not None:
                            sh[nb].wait()
                        gh[nb] = pltpu.async_copy(
                            table.at[isrc.at[j + 2]], bufs[nb], gsems[nb])
                    sh[b] = pltpu.async_copy(
                        bufs[b], acc.at[idst.at[j]], ssems[b], add=True)
                for b in range(NB):
                    if sh[b] is not None:
                        sh[b].wait()
            else:
                sh = [None] * NB
                for j in range(IC):
                    b = j % NB
                    if sh[b] is not None:
                        sh[b].wait()
                    sh[b] = pltpu.async_copy(
                        bufs[0], acc.at[idst.at[j]], ssems[b], add=True)
                for b in range(NB):
                    if sh[b] is not None:
                        sh[b].wait()
            return carry

        lax.fori_loop(0, chunks, chunk, 0)
        plsc.subcore_barrier()
        pltpu.sync_copy(acc.at[pl.ds(lo, rows_per_tile)],
                        out.at[c, pl.ds(lo, rows_per_tile)])

    return agg


def _tc0_body(degs_ref, z_ref, emb_ref, W0_ref, p0_ref, dinv_ref):
    deg = (degs_ref[0, :N, 0:1] + degs_ref[1, :N, 0:1]) + 1.0
    dinv = lax.rsqrt(deg)
    dinv_ref[...] = dinv
    T0 = jnp.dot(emb_ref[...], W0_ref[...],
                 preferred_element_type=jnp.float32)
    z = z_ref[...]
    cls = lax.broadcasted_iota(jnp.int32, (1, MAX_Z), 1)
    oh = (z == cls).astype(jnp.float32)
    h0 = jnp.dot(oh, T0, preferred_element_type=jnp.float32)
    p0_ref[...] = h0 * dinv


def _tc_layer_body(k, S_ref, pprev_ref, dinv_ref, Ws_ref, bs_ref, pk_ref):
    dinv = dinv_ref[...]
    agg = S_ref[0, :N, :] + S_ref[1, :N, :] + pprev_ref[...]
    x = jnp.maximum(dinv * agg + bs_ref[k - 1:k, :], 0.0)
    pk_ref[...] = dinv * jnp.dot(x, Ws_ref[k],
                                 preferred_element_type=jnp.float32)


def _tc_final_body(S_ref, p5_ref, dinv_ref, bs_ref, batch_ref,
                   W1_ref, b1_ref, W2_ref, b2_ref, out_ref):
    dinv = dinv_ref[...]
    agg = S_ref[0, :N, :] + S_ref[1, :N, :] + p5_ref[...]
    x = jnp.maximum(dinv * agg + bs_ref[L - 1:L, :], 0.0)
    seg = lax.broadcasted_iota(jnp.int32, (B, 1), 0)
    ohb = (seg == batch_ref[...]).astype(jnp.float32)
    g = jnp.dot(ohb, x, preferred_element_type=jnp.float32)
    h = jnp.maximum(jnp.dot(g, W1_ref[...],
                            preferred_element_type=jnp.float32)
                    + b1_ref[...], 0.0)
    out_ref[...] = (jnp.dot(h, W2_ref[...],
                            preferred_element_type=jnp.float32)
                    + b2_ref[0:1, 0:1])


def kernel(z, edge_index, batch, emb, Ws, bs, W1, b1, W2, b2):
    E = edge_index.shape[1]
    per_w = -(-E // (NW * 32 * GRP)) * (32 * GRP)
    groups = per_w // GRP
    e_pad = per_w * NW

    ei = edge_index.astype(jnp.int32)
    pad = e_pad - E
    pad_i = jnp.arange(pad, dtype=jnp.int32)
    src_p = jnp.concatenate([ei[0], pad_i % N])
    dst_p = jnp.concatenate([ei[1], TRASH + pad_i % (ACC_ROWS - N)])
    srcs = src_p.reshape(NW, groups, GRP)
    dsts = dst_p.reshape(NW, groups, GRP)

    zeros128 = jnp.zeros((ACC_ROWS, D), jnp.float32)
    ones_blk = jnp.ones((GRP, D), jnp.float32)

    agg128 = _make_agg(groups, gather=True)
    deg_cnt = _make_agg(groups, gather=False)

    degs = deg_cnt(ones_blk, srcs, dsts, zeros128)

    z2 = z.astype(jnp.int32).reshape(N, 1)
    p0, dinv = pl.pallas_call(
        _tc0_body,
        out_shape=(jax.ShapeDtypeStruct((N, D), jnp.float32),
                   jax.ShapeDtypeStruct((N, 1), jnp.float32)),
    )(degs, z2, emb, Ws[0])

    p = p0
    for k in range(1, L + 1):
        S = agg128(p, srcs, dsts, zeros128)
        if k < L:
            p = pl.pallas_call(
                functools.partial(_tc_layer_body, k),
                out_shape=jax.ShapeDtypeStruct((N, D), jnp.float32),
            )(S, p, dinv, Ws, bs)
        else:
            out = pl.pallas_call(
                _tc_final_body,
                out_shape=jax.ShapeDtypeStruct((B, 1), jnp.float32),
            )(S, p, dinv, bs, batch.astype(jnp.int32).reshape(1, N),
              W1, b1.reshape(1, D), W2, b2.reshape(1, 1))

    return out.reshape(-1)
